# baseline (speedup 1.0000x reference)
import jax
import jax.numpy as jnp
from jax import lax
from jax.experimental import pallas as pl
from jax.experimental.pallas import tpu as pltpu
from jax.experimental.pallas import tpu_sc as plsc

NUM_FIELDS = 26
VOCAB = 100000
DIM = 32
BATCH = 16384

NC = 2
NS = 16
NW = NC * NS

SPLIT = 50048
BLEN = VOCAB - SPLIT
QROWS = BATCH // 4


def _sc_body(tbl_hbm, idx_hbm, out_hbm, vec_a, vec_b, ib0, ib1, ib2, out_v,
             asem, bsem, is0, is1, is2, os0, os1, os2, os3):
    d = lax.axis_index("s") * NC + lax.axis_index("c")
    ibufs, isems = (ib0, ib1, ib2), (is0, is1, is2)
    osems = (os0, os1, os2, os3)
    lanes = lax.iota(jnp.int32, 16)

    def _fire_idx(f, q_static, b):
        pltpu.async_copy(
            idx_hbm.at[f, pl.ds(q_static * QROWS, QROWS)], ibufs[b], isems[b]
        )

    def _wait_idx(b):
        pltpu.make_async_copy(
            ibufs[b], out_hbm.at[0, d, pl.ds(0, QROWS)], isems[b]
        ).wait()

    def _drain_out(q):
        pltpu.make_async_copy(
            out_v.at[pl.ds(0, QROWS)], out_hbm.at[0, d, pl.ds(0, QROWS)],
            osems[q],
        ).wait()

    def _pass1(cq, ib):
        @plsc.parallel_loop(0, QROWS, step=16, unroll=8)
        def _p(o):
            iv = ib[pl.ds(o, 16)]
            out_v[pl.ds(cq * QROWS + o, 16)] = plsc.load_gather(vec_a, [iv])

    def _pass2(cq, ib):
        @plsc.parallel_loop(0, QROWS, step=16, unroll=8)
        def _p(o):
            iv = ib[pl.ds(o, 16)]
            m = iv >= SPLIT
            g = plsc.load_gather(vec_b, [iv - SPLIT], mask=m)
            plsc.store_scatter(out_v, [lanes + (cq * QROWS + o)], g, mask=m)

    def _wb(f, q):
        pltpu.async_copy(
            out_v.at[pl.ds(q * QROWS, QROWS)],
            out_hbm.at[f, d, pl.ds(q * QROWS, QROWS)],
            osems[q],
        )

    _fire_idx(0, 0, 0)
    pltpu.async_copy(tbl_hbm.at[0, d, pl.ds(0, SPLIT)], vec_a, asem)

    def _task(f, carry):
        pltpu.make_async_copy(
            vec_a, tbl_hbm.at[0, d, pl.ds(0, SPLIT)], asem
        ).wait()
        pltpu.async_copy(tbl_hbm.at[f, d, pl.ds(SPLIT, BLEN)], vec_b, bsem)
        _fire_idx(f, 1, 1)
        _fire_idx(f, 2, 2)

        for cq, b in ((0, 0), (1, 1), (2, 2), (3, 0)):
            _wait_idx(b)

            @pl.when(f > 0)
            def _():
                _drain_out(cq)

            _pass1(cq, ibufs[b])
            if cq == 0:
                _fire_idx(f, 3, 0)

        pltpu.make_async_copy(
            vec_b, tbl_hbm.at[0, d, pl.ds(SPLIT, BLEN)], bsem
        ).wait()

        @pl.when(f + 1 < NUM_FIELDS)
        def _():
            pltpu.async_copy(
                tbl_hbm.at[f + 1, d, pl.ds(0, SPLIT)], vec_a, asem
            )

        _pass2(3, ib0)
        _wb(f, 3)

        @pl.when(f + 1 < NUM_FIELDS)
        def _():
            _fire_idx(f + 1, 0, 0)

        _pass2(1, ib1)
        _wb(f, 1)
        _fire_idx(f, 0, 1)
        _pass2(2, ib2)
        _wb(f, 2)
        _wait_idx(1)
        _pass2(0, ib1)
        _wb(f, 0)
        return carry

    lax.fori_loop(0, NUM_FIELDS, _task, 0)
    for q in range(4):
        _drain_out(q)


@jax.jit
def _sc_embed(tbl_t, idx_t):
    mesh = plsc.VectorSubcoreMesh(
        core_axis_name="c", subcore_axis_name="s", num_cores=NC, num_subcores=NS
    )
    return pl.kernel(
        _sc_body,
        out_type=jax.ShapeDtypeStruct((NUM_FIELDS, DIM, BATCH), jnp.float32),
        mesh=mesh,
        scratch_types=[
            pltpu.VMEM((SPLIT,), jnp.float32),
            pltpu.VMEM((BLEN,), jnp.float32),
            pltpu.VMEM((QROWS,), jnp.int32),
            pltpu.VMEM((QROWS,), jnp.int32),
            pltpu.VMEM((QROWS,), jnp.int32),
            pltpu.VMEM((BATCH,), jnp.float32),
            pltpu.SemaphoreType.DMA,
            pltpu.SemaphoreType.DMA,
            pltpu.SemaphoreType.DMA,
            pltpu.SemaphoreType.DMA,
            pltpu.SemaphoreType.DMA,
            pltpu.SemaphoreType.DMA,
            pltpu.SemaphoreType.DMA,
            pltpu.SemaphoreType.DMA,
            pltpu.SemaphoreType.DMA,
        ],
        compiler_params=pltpu.CompilerParams(
            use_tc_tiling_on_sc=True, needs_layout_passes=False
        ),
    )(tbl_t, idx_t)


def kernel(sparse_inputs, tables):
    idx_t = jnp.transpose(sparse_inputs.astype(jnp.int32))
    tbl_t = jnp.transpose(tables, (0, 2, 1))
    out_t = _sc_embed(tbl_t, idx_t)
    return jnp.transpose(out_t, (2, 0, 1))


# speedup vs baseline: 1.2279x; 1.2279x over previous
import jax
import jax.numpy as jnp
from jax import lax
from jax.experimental import pallas as pl
from jax.experimental.pallas import tpu as pltpu
from jax.experimental.pallas import tpu_sc as plsc

NUM_FIELDS = 26
VOCAB = 100000
DIM = 32
BATCH = 16384

NC = 2
NS = 16
NW = NC * NS

SPLIT = 50048
BLEN = VOCAB - SPLIT
QROWS = BATCH // 4


def _sc_body(tbl_hbm, idx_hbm, out_hbm, vec_a, vec_b, ib0, ib1, ib2, out_v,
             ish, asem, bsem, ssem, is0, is1, is2, os0, os1, os2, os3):
    sid = lax.axis_index("s")
    d = sid * NC + lax.axis_index("c")
    ibufs, isems = (ib0, ib1, ib2), (is0, is1, is2)
    osems = (os0, os1, os2, os3)
    lanes = lax.iota(jnp.int32, 16)

    def _fire_idx(f, q_static, b):
        pltpu.async_copy(
            ish.at[lax.rem(f, 2), pl.ds(q_static * QROWS, QROWS)],
            ibufs[b], isems[b],
        )

    def _wait_idx(b):
        pltpu.make_async_copy(
            ibufs[b], out_hbm.at[0, d, pl.ds(0, QROWS)], isems[b]
        ).wait()

    def _drain_out(q):
        pltpu.make_async_copy(
            out_v.at[pl.ds(0, QROWS)], out_hbm.at[0, d, pl.ds(0, QROWS)],
            osems[q],
        ).wait()

    def _pass1(cq, ib):
        @plsc.parallel_loop(0, QROWS, step=16, unroll=8)
        def _p(o):
            iv = ib[pl.ds(o, 16)]
            out_v[pl.ds(cq * QROWS + o, 16)] = plsc.load_gather(vec_a, [iv])

    def _pass2(cq, ib):
        @plsc.parallel_loop(0, QROWS, step=16, unroll=8)
        def _p(o):
            iv = ib[pl.ds(o, 16)]
            m = iv >= SPLIT
            g = plsc.load_gather(vec_b, [iv - SPLIT], mask=m)
            plsc.store_scatter(out_v, [lanes + (cq * QROWS + o)], g, mask=m)

    def _wb(f, q):
        pltpu.async_copy(
            out_v.at[pl.ds(q * QROWS, QROWS)],
            out_hbm.at[f, d, pl.ds(q * QROWS, QROWS)],
            osems[q],
        )

    @pl.when(sid == 0)
    def _():
        pltpu.async_copy(idx_hbm.at[0], ish.at[0], ssem)

    pltpu.async_copy(tbl_hbm.at[0, d, pl.ds(0, SPLIT)], vec_a, asem)

    def _task(f, carry):
        @pl.when(sid == 0)
        def _():
            pltpu.make_async_copy(idx_hbm.at[0], ish.at[0], ssem).wait()

        plsc.subcore_barrier()

        @pl.when((sid == 0) & (f + 1 < NUM_FIELDS))
        def _():
            pltpu.async_copy(idx_hbm.at[f + 1], ish.at[lax.rem(f + 1, 2)], ssem)

        pltpu.make_async_copy(
            vec_a, tbl_hbm.at[0, d, pl.ds(0, SPLIT)], asem
        ).wait()
        pltpu.async_copy(tbl_hbm.at[f, d, pl.ds(SPLIT, BLEN)], vec_b, bsem)
        _fire_idx(f, 0, 0)
        _fire_idx(f, 1, 1)
        _fire_idx(f, 2, 2)

        for cq, b in ((0, 0), (1, 1), (2, 2), (3, 0)):
            _wait_idx(b)

            @pl.when(f > 0)
            def _():
                _drain_out(cq)

            _pass1(cq, ibufs[b])
            if cq == 0:
                _fire_idx(f, 3, 0)

        pltpu.make_async_copy(
            vec_b, tbl_hbm.at[0, d, pl.ds(SPLIT, BLEN)], bsem
        ).wait()

        @pl.when(f + 1 < NUM_FIELDS)
        def _():
            pltpu.async_copy(
                tbl_hbm.at[f + 1, d, pl.ds(0, SPLIT)], vec_a, asem
            )

        _pass2(3, ib0)
        _wb(f, 3)
        _pass2(1, ib1)
        _wb(f, 1)
        _fire_idx(f, 0, 1)
        _pass2(2, ib2)
        _wb(f, 2)
        _wait_idx(1)
        _pass2(0, ib1)
        _wb(f, 0)
        return carry

    lax.fori_loop(0, NUM_FIELDS, _task, 0)
    for q in range(4):
        _drain_out(q)


@jax.jit
def _sc_embed(tbl_t, idx_t):
    mesh = plsc.VectorSubcoreMesh(
        core_axis_name="c", subcore_axis_name="s", num_cores=NC, num_subcores=NS
    )
    return pl.kernel(
        _sc_body,
        out_type=jax.ShapeDtypeStruct((NUM_FIELDS, DIM, BATCH), jnp.float32),
        mesh=mesh,
        scratch_types=[
            pltpu.VMEM((SPLIT,), jnp.float32),
            pltpu.VMEM((BLEN,), jnp.float32),
            pltpu.VMEM((QROWS,), jnp.int32),
            pltpu.VMEM((QROWS,), jnp.int32),
            pltpu.VMEM((QROWS,), jnp.int32),
            pltpu.VMEM((BATCH,), jnp.float32),
            pltpu.VMEM_SHARED((2, BATCH), jnp.int32),
            pltpu.SemaphoreType.DMA,
            pltpu.SemaphoreType.DMA,
            pltpu.SemaphoreType.DMA,
            pltpu.SemaphoreType.DMA,
            pltpu.SemaphoreType.DMA,
            pltpu.SemaphoreType.DMA,
            pltpu.SemaphoreType.DMA,
            pltpu.SemaphoreType.DMA,
            pltpu.SemaphoreType.DMA,
            pltpu.SemaphoreType.DMA,
        ],
        compiler_params=pltpu.CompilerParams(
            use_tc_tiling_on_sc=True, needs_layout_passes=False
        ),
    )(tbl_t, idx_t)


def kernel(sparse_inputs, tables):
    idx_t = jnp.transpose(sparse_inputs.astype(jnp.int32))
    tbl_t = jnp.transpose(tables, (0, 2, 1))
    out_t = _sc_embed(tbl_t, idx_t)
    return jnp.transpose(out_t, (2, 0, 1))


# speedup vs baseline: 1.2817x; 1.0438x over previous
import jax
import jax.numpy as jnp
from jax import lax
from jax.experimental import pallas as pl
from jax.experimental.pallas import tpu as pltpu
from jax.experimental.pallas import tpu_sc as plsc

NUM_FIELDS = 26
VOCAB = 100000
DIM = 32
BATCH = 16384

NC = 2
NS = 16
NW = NC * NS

SPLIT = 41856
BLEN = VOCAB - SPLIT
QROWS = BATCH // 4


def _sc_body(tbl_hbm, idx_hbm, out_hbm, vec_a, vec_b, ib0, ib1, ib2, out_v,
             ish, asem, bsem, ssem, is0, is1, is2, os0, os1, os2, os3):
    sid = lax.axis_index("s")
    d = sid * NC + lax.axis_index("c")
    ibufs, isems = (ib0, ib1, ib2), (is0, is1, is2)
    osems = (os0, os1, os2, os3)
    lanes = lax.iota(jnp.int32, 16)

    def _fire_idx(f, q_static, b):
        pltpu.async_copy(
            ish.at[lax.rem(f, 2), pl.ds(q_static * QROWS, QROWS)],
            ibufs[b], isems[b],
        )

    def _wait_idx(b):
        pltpu.make_async_copy(
            ibufs[b], out_hbm.at[0, d, pl.ds(0, QROWS)], isems[b]
        ).wait()

    def _drain_out(q):
        pltpu.make_async_copy(
            out_v.at[pl.ds(0, QROWS)], out_hbm.at[0, d, pl.ds(0, QROWS)],
            osems[q],
        ).wait()

    def _pass1(cq, ib):
        @plsc.parallel_loop(0, QROWS, step=16, unroll=8)
        def _p(o):
            iv = ib[pl.ds(o, 16)]
            out_v[pl.ds(cq * QROWS + o, 16)] = plsc.load_gather(vec_a, [iv])

    def _pass2(cq, ib):
        @plsc.parallel_loop(0, QROWS, step=16, unroll=8)
        def _p(o):
            iv = ib[pl.ds(o, 16)]
            m = iv >= SPLIT
            g = plsc.load_gather(vec_b, [iv - SPLIT], mask=m)
            plsc.store_scatter(out_v, [lanes + (cq * QROWS + o)], g, mask=m)

    def _wb(f, q):
        pltpu.async_copy(
            out_v.at[pl.ds(q * QROWS, QROWS)],
            out_hbm.at[f, d, pl.ds(q * QROWS, QROWS)],
            osems[q],
        )

    @pl.when(sid == 0)
    def _():
        pltpu.async_copy(idx_hbm.at[0], ish.at[0], ssem)

    pltpu.async_copy(tbl_hbm.at[0, d, pl.ds(0, SPLIT)], vec_a, asem)

    def _task(f, carry):
        @pl.when(sid == 0)
        def _():
            pltpu.make_async_copy(idx_hbm.at[0], ish.at[0], ssem).wait()

        plsc.subcore_barrier()

        @pl.when((sid == 0) & (f + 1 < NUM_FIELDS))
        def _():
            pltpu.async_copy(idx_hbm.at[f + 1], ish.at[lax.rem(f + 1, 2)], ssem)

        pltpu.make_async_copy(
            vec_a, tbl_hbm.at[0, d, pl.ds(0, SPLIT)], asem
        ).wait()
        pltpu.async_copy(tbl_hbm.at[f, d, pl.ds(SPLIT, BLEN)], vec_b, bsem)
        _fire_idx(f, 0, 0)
        _fire_idx(f, 1, 1)
        _fire_idx(f, 2, 2)

        for cq, b in ((0, 0), (1, 1), (2, 2), (3, 0)):
            _wait_idx(b)

            @pl.when(f > 0)
            def _():
                _drain_out(cq)

            _pass1(cq, ibufs[b])
            if cq == 0:
                _fire_idx(f, 3, 0)

        pltpu.make_async_copy(
            vec_b, tbl_hbm.at[0, d, pl.ds(SPLIT, BLEN)], bsem
        ).wait()

        @pl.when(f + 1 < NUM_FIELDS)
        def _():
            pltpu.async_copy(
                tbl_hbm.at[f + 1, d, pl.ds(0, SPLIT)], vec_a, asem
            )

        _pass2(3, ib0)
        _wb(f, 3)
        _pass2(1, ib1)
        _wb(f, 1)
        _fire_idx(f, 0, 1)
        _pass2(2, ib2)
        _wb(f, 2)
        _wait_idx(1)
        _pass2(0, ib1)
        _wb(f, 0)
        return carry

    lax.fori_loop(0, NUM_FIELDS, _task, 0)
    for q in range(4):
        _drain_out(q)


@jax.jit
def _sc_embed(tbl_t, idx_t):
    mesh = plsc.VectorSubcoreMesh(
        core_axis_name="c", subcore_axis_name="s", num_cores=NC, num_subcores=NS
    )
    return pl.kernel(
        _sc_body,
        out_type=jax.ShapeDtypeStruct((NUM_FIELDS, DIM, BATCH), jnp.float32),
        mesh=mesh,
        scratch_types=[
            pltpu.VMEM((SPLIT,), jnp.float32),
            pltpu.VMEM((BLEN,), jnp.float32),
            pltpu.VMEM((QROWS,), jnp.int32),
            pltpu.VMEM((QROWS,), jnp.int32),
            pltpu.VMEM((QROWS,), jnp.int32),
            pltpu.VMEM((BATCH,), jnp.float32),
            pltpu.VMEM_SHARED((2, BATCH), jnp.int32),
            pltpu.SemaphoreType.DMA,
            pltpu.SemaphoreType.DMA,
            pltpu.SemaphoreType.DMA,
            pltpu.SemaphoreType.DMA,
            pltpu.SemaphoreType.DMA,
            pltpu.SemaphoreType.DMA,
            pltpu.SemaphoreType.DMA,
            pltpu.SemaphoreType.DMA,
            pltpu.SemaphoreType.DMA,
            pltpu.SemaphoreType.DMA,
        ],
        compiler_params=pltpu.CompilerParams(
            use_tc_tiling_on_sc=True, needs_layout_passes=False
        ),
    )(tbl_t, idx_t)


def kernel(sparse_inputs, tables):
    idx_t = jnp.transpose(sparse_inputs.astype(jnp.int32))
    tbl_t = jnp.transpose(tables, (0, 2, 1))
    out_t = _sc_embed(tbl_t, idx_t)
    return jnp.transpose(out_t, (2, 0, 1))
